# baseline (speedup 1.0000x reference)
import functools

import jax
import jax.numpy as jnp
from jax import lax
from jax.experimental import pallas as pl
from jax.experimental.pallas import tpu as pltpu
from jax.experimental.pallas import tpu_sc as plsc

_BS = 4096
_S = 16
_E = 8
_D = 64
_L = 16
_DHI = 8
_DLO = 8
_TB = 32
_BL = 128
_G = _BL // _L

_K_SC = 4
_K_TC = _TB - _K_SC
_GPB = 32 // _K_SC
_S_W = _S // _GPB
_NCH = _S_W * _DHI


def _tc_body(mask, edge, ent, self_, nv_out, sv_out):
    m = [mask[:, e, 0, 0, :] for e in range(_E)]
    cnt = m[0]
    for e in range(1, _E):
        cnt = cnt + m[e]
    inv = 1.0 / jnp.where(cnt == 0.0, 1.0, cnt)
    w = [m[e] * inv for e in range(_E)]
    agg = edge[:, 0] * w[0][:, None, :]
    for e in range(1, _E):
        agg = agg + edge[:, e] * w[e][:, None, :]
    nv = ent[...] + agg
    nv_out[...] = nv
    sacc = nv[0]
    for s in range(1, _S):
        sacc = sacc + nv[s]
    sv_out[...] = self_[...] + sacc * (1.0 / _S)


def _tc_call(mask5, edge_t, ent_t, self_t):
    return pl.pallas_call(
        _tc_body,
        grid=(_K_TC,),
        in_specs=[
            pl.BlockSpec((_S, _E, 1, 1, _BL), lambda i: (0, 0, i, 0, 0)),
            pl.BlockSpec((_S, _E, _D, _BL), lambda i: (0, 0, 0, i)),
            pl.BlockSpec((_S, _D, _BL), lambda i: (0, 0, i)),
            pl.BlockSpec((_D, _BL), lambda i: (0, i)),
        ],
        out_specs=[
            pl.BlockSpec((_S, _D, _BL), lambda i: (0, 0, i)),
            pl.BlockSpec((_D, _BL), lambda i: (0, i)),
        ],
        out_shape=[
            jax.ShapeDtypeStruct((_S, _D, _BS), jnp.float32),
            jax.ShapeDtypeStruct((_D, _BS), jnp.float32),
        ],
    )(mask5, edge_t, ent_t, self_t)


def _sc_kernel(edge6, mask3, ent5,
               nv_out, psv_out,
               ebuf0, ebuf1, entb0, entb1, obuf0, obuf1,
               mbuf, invbuf, svacc,
               si0, si1, so0, so1, sm):
    info = plsc.get_sparse_core_info()
    nc = info.num_cores
    w = lax.axis_index("s") * nc + lax.axis_index("c")
    tbl = w // _GPB
    wg = lax.rem(w, _GPB)
    tb = _K_TC + tbl
    s_base = wg * _S_W

    pltpu.async_copy(mask3.at[pl.ds(s_base * _E, _S_W * _E), tb], mbuf, sm)
    pltpu.make_async_copy(
        mask3.at[pl.ds(s_base * _E, _S_W * _E), tb], mbuf, sm).wait()

    for sl in range(_S_W):
        for g in range(_G):
            cnt = mbuf[sl * _E, pl.ds(g * _L, _L)]
            for e in range(1, _E):
                cnt = cnt + mbuf[sl * _E + e, pl.ds(g * _L, _L)]
            invbuf[sl, pl.ds(g * _L, _L)] = 1.0 / jnp.where(cnt == 0.0, 1.0, cnt)

    def zbody(i, c):
        for g in range(_G):
            svacc[i, pl.ds(g * _L, _L)] = jnp.zeros((_L,), jnp.float32)
        return c
    lax.fori_loop(0, _DHI * _DLO, zbody, 0)

    def start_in(sg, dhi, ebuf, entb, si):
        pltpu.async_copy(edge6.at[sg, :, dhi, tb], ebuf, si)
        pltpu.async_copy(ent5.at[sg, dhi, tb], entb, si)

    def wait_in(sg, dhi, ebuf, entb, si):
        pltpu.make_async_copy(edge6.at[sg, :, dhi, tb], ebuf, si).wait()
        pltpu.make_async_copy(ent5.at[sg, dhi, tb], entb, si).wait()

    def compute_chunk(sl, dhi, ebuf, entb, obuf):
        for g in range(_G):
            inv = invbuf[sl, pl.ds(g * _L, _L)]
            wv = [mbuf[sl * _E + e, pl.ds(g * _L, _L)] * inv for e in range(_E)]
            for dlo in range(_DLO):
                p = [wv[e] * ebuf[e, dlo, pl.ds(g * _L, _L)]
                     for e in range(_E)]
                q = [p[0] + p[1], p[2] + p[3], p[4] + p[5], p[6] + p[7]]
                r = [q[0] + q[1], q[2] + q[3]]
                nv = (entb[dlo, pl.ds(g * _L, _L)] + r[0]) + r[1]
                obuf[dlo, pl.ds(g * _L, _L)] = nv
                plsc.addupdate(
                    svacc.at[dhi * _DLO + dlo, pl.ds(g * _L, _L)], nv)

    def chunk_s(k):
        return s_base + k // _DHI, k // _DHI

    start_in(s_base, 0, ebuf0, entb0, si0)
    start_in(s_base, 1, ebuf1, entb1, si1)

    def pair_body(j, carry):
        k0 = 2 * j
        k1 = k0 + 1
        s0g, s0l = chunk_s(k0)
        d0 = lax.rem(k0, _DHI)
        s1g, s1l = chunk_s(k1)
        d1 = lax.rem(k1, _DHI)

        wait_in(s0g, d0, ebuf0, entb0, si0)

        @pl.when(j >= 1)
        def _():
            km = k0 - 2
            pltpu.make_async_copy(
                obuf0, nv_out.at[s_base + km // _DHI, lax.rem(km, _DHI), tbl],
                so0).wait()

        compute_chunk(s0l, d0, ebuf0, entb0, obuf0)
        pltpu.async_copy(obuf0, nv_out.at[s0g, d0, tbl], so0)

        @pl.when(k0 + 2 < _NCH)
        def _():
            kn = k0 + 2
            start_in(s_base + kn // _DHI, lax.rem(kn, _DHI), ebuf0, entb0, si0)

        wait_in(s1g, d1, ebuf1, entb1, si1)

        @pl.when(j >= 1)
        def _():
            km = k1 - 2
            pltpu.make_async_copy(
                obuf1, nv_out.at[s_base + km // _DHI, lax.rem(km, _DHI), tbl],
                so1).wait()

        compute_chunk(s1l, d1, ebuf1, entb1, obuf1)
        pltpu.async_copy(obuf1, nv_out.at[s1g, d1, tbl], so1)

        @pl.when(k1 + 2 < _NCH)
        def _():
            kn = k1 + 2
            start_in(s_base + kn // _DHI, lax.rem(kn, _DHI), ebuf1, entb1, si1)

        return carry

    lax.fori_loop(0, _NCH // 2, pair_body, 0)

    pltpu.make_async_copy(
        obuf0, nv_out.at[s_base + _S_W - 1, _DHI - 2, tbl], so0).wait()
    pltpu.make_async_copy(
        obuf1, nv_out.at[s_base + _S_W - 1, _DHI - 1, tbl], so1).wait()

    pltpu.async_copy(svacc, psv_out.at[wg, tbl], sm)
    pltpu.make_async_copy(svacc, psv_out.at[wg, tbl], sm).wait()


def _sc_call(edge6, mask3, ent5):
    mesh = plsc.VectorSubcoreMesh(core_axis_name="c", subcore_axis_name="s")
    body = functools.partial(
        pl.kernel,
        mesh=mesh,
        out_type=(
            jax.ShapeDtypeStruct((_S, _DHI, _K_SC, _DLO, _BL), jnp.float32),
            jax.ShapeDtypeStruct((_GPB, _K_SC, _DHI * _DLO, _BL), jnp.float32),
        ),
        scratch_types=[
            pltpu.VMEM((_E, _DLO, _BL), jnp.float32),
            pltpu.VMEM((_E, _DLO, _BL), jnp.float32),
            pltpu.VMEM((_DLO, _BL), jnp.float32),
            pltpu.VMEM((_DLO, _BL), jnp.float32),
            pltpu.VMEM((_DLO, _BL), jnp.float32),
            pltpu.VMEM((_DLO, _BL), jnp.float32),
            pltpu.VMEM((_S_W * _E, _BL), jnp.float32),
            pltpu.VMEM((_S_W, _BL), jnp.float32),
            pltpu.VMEM((_DHI * _DLO, _BL), jnp.float32),
            pltpu.SemaphoreType.DMA,
            pltpu.SemaphoreType.DMA,
            pltpu.SemaphoreType.DMA,
            pltpu.SemaphoreType.DMA,
            pltpu.SemaphoreType.DMA,
        ],
    )(_sc_kernel)
    return body(edge6, mask3, ent5)


def _ep_body(nv_in, sv_in, nvsc, psv, self_, nv_out, sv_out):
    del nv_in, sv_in
    for dhi in range(_DHI):
        nv_out[:, dhi * _DLO:(dhi + 1) * _DLO, :] = nvsc[:, dhi, 0]
    ps = psv[0, 0]
    for g in range(1, _GPB):
        ps = ps + psv[g, 0]
    sv_out[...] = self_[...] + ps.reshape(_D, _BL) * (1.0 / _S)


def _ep_call(nv_full, sv_full, nv_sc, psv, self_t):
    return pl.pallas_call(
        _ep_body,
        grid=(_K_SC,),
        in_specs=[
            pl.BlockSpec((_S, _D, _BL), lambda j: (0, 0, _K_TC + j)),
            pl.BlockSpec((_D, _BL), lambda j: (0, _K_TC + j)),
            pl.BlockSpec((_S, _DHI, 1, _DLO, _BL), lambda j: (0, 0, j, 0, 0)),
            pl.BlockSpec((_GPB, 1, _DHI * _DLO, _BL), lambda j: (0, j, 0, 0)),
            pl.BlockSpec((_D, _BL), lambda j: (0, _K_TC + j)),
        ],
        out_specs=[
            pl.BlockSpec((_S, _D, _BL), lambda j: (0, 0, _K_TC + j)),
            pl.BlockSpec((_D, _BL), lambda j: (0, _K_TC + j)),
        ],
        out_shape=[
            jax.ShapeDtypeStruct((_S, _D, _BS), jnp.float32),
            jax.ShapeDtypeStruct((_D, _BS), jnp.float32),
        ],
        input_output_aliases={0: 0, 1: 1},
    )(nv_full, sv_full, nv_sc, psv, self_t)


@jax.jit
def _run(mask3, edge_t, ent_t, self_t, edge6, ent5):
    nv_sc, psv = _sc_call(edge6, mask3, ent5)
    nv_full, sv_full = _tc_call(mask3.reshape(_S, _E, _TB, 1, _BL),
                                edge_t, ent_t, self_t)
    return _ep_call(nv_full, sv_full, nv_sc, psv, self_t)


def kernel(self_vectors, neighbor_entity_vectors, neighbor_edge_vectors, masks):
    bs = self_vectors.shape[0]
    edge_t = neighbor_edge_vectors.reshape(bs, _S, _E, _D).transpose(1, 2, 3, 0)
    ent_t = neighbor_entity_vectors.reshape(bs, _S, _D).transpose(1, 2, 0)
    self_t = self_vectors.reshape(bs, _D).transpose(1, 0)
    edge6 = edge_t.reshape(_S, _E, _DHI, _DLO, _TB, _BL).transpose(0, 1, 2, 4, 3, 5)
    mask3 = masks.reshape(bs, _S, _E).transpose(1, 2, 0).reshape(_S * _E, _TB, _BL)
    ent5 = ent_t.reshape(_S, _DHI, _DLO, _TB, _BL).transpose(0, 1, 3, 2, 4)
    nv_t, sv_t = _run(mask3, edge_t, ent_t, self_t, edge6, ent5)
    nv = nv_t.transpose(2, 0, 1).reshape(bs, 1, _S, _D)
    sv = sv_t.transpose(1, 0).reshape(bs, 1, _D)
    return (sv, nv)


# speedup vs baseline: 1.0477x; 1.0477x over previous
import functools

import jax
import jax.numpy as jnp
from jax import lax
from jax.experimental import pallas as pl
from jax.experimental.pallas import tpu as pltpu
from jax.experimental.pallas import tpu_sc as plsc

_BS = 4096
_S = 16
_E = 8
_D = 64
_L = 16
_DHI = 8
_DLO = 8
_TB = 32
_BL = 128
_G = _BL // _L

_K_SC = 4
_K_TC = _TB - _K_SC
_GPB = 32 // _K_SC
_S_W = _S // _GPB
_NCH = _S_W * _DHI


def _tc_body(mask, edge, ent, self_, nv_out, sv_out):
    m = [mask[:, e, :] for e in range(_E)]
    cnt = m[0]
    for e in range(1, _E):
        cnt = cnt + m[e]
    inv = 1.0 / jnp.where(cnt == 0.0, 1.0, cnt)
    w = [m[e] * inv for e in range(_E)]
    agg = edge[:, 0] * w[0][:, None, :]
    for e in range(1, _E):
        agg = agg + edge[:, e] * w[e][:, None, :]
    nv = ent[...] + agg
    nv_out[...] = nv
    sacc = nv[0]
    for s in range(1, _S):
        sacc = sacc + nv[s]
    sv_out[...] = self_[...] + sacc * (1.0 / _S)


def _tc_call(mask_t, edge_t, ent_t, self_t):
    return pl.pallas_call(
        _tc_body,
        grid=(_K_TC,),
        in_specs=[
            pl.BlockSpec((_S, _E, _BL), lambda i: (0, 0, i)),
            pl.BlockSpec((_S, _E, _D, _BL), lambda i: (0, 0, 0, i)),
            pl.BlockSpec((_S, _D, _BL), lambda i: (0, 0, i)),
            pl.BlockSpec((_D, _BL), lambda i: (0, i)),
        ],
        out_specs=[
            pl.BlockSpec((_S, _D, _BL), lambda i: (0, 0, i)),
            pl.BlockSpec((_D, _BL), lambda i: (0, i)),
        ],
        out_shape=[
            jax.ShapeDtypeStruct((_S, _D, _BS), jnp.float32),
            jax.ShapeDtypeStruct((_D, _BS), jnp.float32),
        ],
    )(mask_t, edge_t, ent_t, self_t)


def _sc_kernel(edge6, mask4, ent5,
               nv_out, psv_out,
               ebuf0, ebuf1, entb0, entb1, obuf0, obuf1,
               mbuf, invbuf, svacc,
               si0, si1, so0, so1, sm):
    info = plsc.get_sparse_core_info()
    nc = info.num_cores
    w = lax.axis_index("s") * nc + lax.axis_index("c")
    tbl = w // _GPB
    wg = lax.rem(w, _GPB)
    tb = _K_TC + tbl
    s_base = wg * _S_W

    pltpu.async_copy(mask4.at[pl.ds(s_base, _S_W), :, tb], mbuf, sm)
    pltpu.make_async_copy(
        mask4.at[pl.ds(s_base, _S_W), :, tb], mbuf, sm).wait()

    for sl in range(_S_W):
        for g in range(_G):
            cnt = mbuf[sl, 0, pl.ds(g * _L, _L)]
            for e in range(1, _E):
                cnt = cnt + mbuf[sl, e, pl.ds(g * _L, _L)]
            invbuf[sl, pl.ds(g * _L, _L)] = 1.0 / jnp.where(cnt == 0.0, 1.0, cnt)

    def zbody(i, c):
        for g in range(_G):
            svacc[i, pl.ds(g * _L, _L)] = jnp.zeros((_L,), jnp.float32)
        return c
    lax.fori_loop(0, _DHI * _DLO, zbody, 0)

    def start_in(sg, dhi, ebuf, entb, si):
        pltpu.async_copy(edge6.at[sg, :, dhi, tb], ebuf, si)
        pltpu.async_copy(ent5.at[sg, dhi, tb], entb, si)

    def wait_in(sg, dhi, ebuf, entb, si):
        pltpu.make_async_copy(edge6.at[sg, :, dhi, tb], ebuf, si).wait()
        pltpu.make_async_copy(ent5.at[sg, dhi, tb], entb, si).wait()

    def compute_chunk(sl, dhi, ebuf, entb, obuf):
        for g in range(_G):
            inv = invbuf[sl, pl.ds(g * _L, _L)]
            wv = [mbuf[sl, e, pl.ds(g * _L, _L)] * inv for e in range(_E)]
            for dlo in range(_DLO):
                p = [wv[e] * ebuf[e, dlo, pl.ds(g * _L, _L)]
                     for e in range(_E)]
                q = [p[0] + p[1], p[2] + p[3], p[4] + p[5], p[6] + p[7]]
                r = [q[0] + q[1], q[2] + q[3]]
                nv = (entb[dlo, pl.ds(g * _L, _L)] + r[0]) + r[1]
                obuf[dlo, pl.ds(g * _L, _L)] = nv
                plsc.addupdate(
                    svacc.at[dhi * _DLO + dlo, pl.ds(g * _L, _L)], nv)

    def chunk_s(k):
        return s_base + k // _DHI, k // _DHI

    start_in(s_base, 0, ebuf0, entb0, si0)
    start_in(s_base, 1, ebuf1, entb1, si1)

    def pair_body(j, carry):
        k0 = 2 * j
        k1 = k0 + 1
        s0g, s0l = chunk_s(k0)
        d0 = lax.rem(k0, _DHI)
        s1g, s1l = chunk_s(k1)
        d1 = lax.rem(k1, _DHI)

        wait_in(s0g, d0, ebuf0, entb0, si0)

        @pl.when(j >= 1)
        def _():
            km = k0 - 2
            pltpu.make_async_copy(
                obuf0, nv_out.at[s_base + km // _DHI, lax.rem(km, _DHI), tbl],
                so0).wait()

        compute_chunk(s0l, d0, ebuf0, entb0, obuf0)
        pltpu.async_copy(obuf0, nv_out.at[s0g, d0, tbl], so0)

        @pl.when(k0 + 2 < _NCH)
        def _():
            kn = k0 + 2
            start_in(s_base + kn // _DHI, lax.rem(kn, _DHI), ebuf0, entb0, si0)

        wait_in(s1g, d1, ebuf1, entb1, si1)

        @pl.when(j >= 1)
        def _():
            km = k1 - 2
            pltpu.make_async_copy(
                obuf1, nv_out.at[s_base + km // _DHI, lax.rem(km, _DHI), tbl],
                so1).wait()

        compute_chunk(s1l, d1, ebuf1, entb1, obuf1)
        pltpu.async_copy(obuf1, nv_out.at[s1g, d1, tbl], so1)

        @pl.when(k1 + 2 < _NCH)
        def _():
            kn = k1 + 2
            start_in(s_base + kn // _DHI, lax.rem(kn, _DHI), ebuf1, entb1, si1)

        return carry

    lax.fori_loop(0, _NCH // 2, pair_body, 0)

    pltpu.make_async_copy(
        obuf0, nv_out.at[s_base + _S_W - 1, _DHI - 2, tbl], so0).wait()
    pltpu.make_async_copy(
        obuf1, nv_out.at[s_base + _S_W - 1, _DHI - 1, tbl], so1).wait()

    pltpu.async_copy(svacc, psv_out.at[wg, tbl], sm)
    pltpu.make_async_copy(svacc, psv_out.at[wg, tbl], sm).wait()


def _sc_call(edge6, mask4, ent5):
    mesh = plsc.VectorSubcoreMesh(core_axis_name="c", subcore_axis_name="s")
    body = functools.partial(
        pl.kernel,
        mesh=mesh,
        out_type=(
            jax.ShapeDtypeStruct((_S, _DHI, _K_SC, _DLO, _BL), jnp.float32),
            jax.ShapeDtypeStruct((_GPB, _K_SC, _DHI * _DLO, _BL), jnp.float32),
        ),
        scratch_types=[
            pltpu.VMEM((_E, _DLO, _BL), jnp.float32),
            pltpu.VMEM((_E, _DLO, _BL), jnp.float32),
            pltpu.VMEM((_DLO, _BL), jnp.float32),
            pltpu.VMEM((_DLO, _BL), jnp.float32),
            pltpu.VMEM((_DLO, _BL), jnp.float32),
            pltpu.VMEM((_DLO, _BL), jnp.float32),
            pltpu.VMEM((_S_W, _E, _BL), jnp.float32),
            pltpu.VMEM((_S_W, _BL), jnp.float32),
            pltpu.VMEM((_DHI * _DLO, _BL), jnp.float32),
            pltpu.SemaphoreType.DMA,
            pltpu.SemaphoreType.DMA,
            pltpu.SemaphoreType.DMA,
            pltpu.SemaphoreType.DMA,
            pltpu.SemaphoreType.DMA,
        ],
    )(_sc_kernel)
    return body(edge6, mask4, ent5)


def _ep_body(nv_in, sv_in, nvsc, psv, self_, nv_out, sv_out):
    del nv_in, sv_in
    for dhi in range(_DHI):
        nv_out[:, dhi * _DLO:(dhi + 1) * _DLO, :] = nvsc[:, dhi, 0]
    ps = psv[0, 0]
    for g in range(1, _GPB):
        ps = ps + psv[g, 0]
    sv_out[...] = self_[...] + ps.reshape(_D, _BL) * (1.0 / _S)


def _ep_call(nv_full, sv_full, nv_sc, psv, self_t):
    return pl.pallas_call(
        _ep_body,
        grid=(_K_SC,),
        in_specs=[
            pl.BlockSpec((_S, _D, _BL), lambda j: (0, 0, _K_TC + j)),
            pl.BlockSpec((_D, _BL), lambda j: (0, _K_TC + j)),
            pl.BlockSpec((_S, _DHI, 1, _DLO, _BL), lambda j: (0, 0, j, 0, 0)),
            pl.BlockSpec((_GPB, 1, _DHI * _DLO, _BL), lambda j: (0, j, 0, 0)),
            pl.BlockSpec((_D, _BL), lambda j: (0, _K_TC + j)),
        ],
        out_specs=[
            pl.BlockSpec((_S, _D, _BL), lambda j: (0, 0, _K_TC + j)),
            pl.BlockSpec((_D, _BL), lambda j: (0, _K_TC + j)),
        ],
        out_shape=[
            jax.ShapeDtypeStruct((_S, _D, _BS), jnp.float32),
            jax.ShapeDtypeStruct((_D, _BS), jnp.float32),
        ],
        input_output_aliases={0: 0, 1: 1},
    )(nv_full, sv_full, nv_sc, psv, self_t)


@jax.jit
def _run(mask_t, mask4, edge_t, ent_t, self_t, edge6, ent5):
    nv_sc, psv = _sc_call(edge6, mask4, ent5)
    nv_full, sv_full = _tc_call(mask_t, edge_t, ent_t, self_t)
    return _ep_call(nv_full, sv_full, nv_sc, psv, self_t)


def kernel(self_vectors, neighbor_entity_vectors, neighbor_edge_vectors, masks):
    bs = self_vectors.shape[0]
    edge_t = neighbor_edge_vectors.reshape(bs, _S, _E, _D).transpose(1, 2, 3, 0)
    ent_t = neighbor_entity_vectors.reshape(bs, _S, _D).transpose(1, 2, 0)
    self_t = self_vectors.reshape(bs, _D).transpose(1, 0)
    edge6 = edge_t.reshape(_S, _E, _DHI, _DLO, _TB, _BL).transpose(0, 1, 2, 4, 3, 5)
    mask_t = masks.reshape(bs, _S, _E).transpose(1, 2, 0)
    mask4 = mask_t.reshape(_S, _E, _TB, _BL)
    ent5 = ent_t.reshape(_S, _DHI, _DLO, _TB, _BL).transpose(0, 1, 3, 2, 4)
    nv_t, sv_t = _run(mask_t, mask4, edge_t, ent_t, self_t, edge6, ent5)
    nv = nv_t.transpose(2, 0, 1).reshape(bs, 1, _S, _D)
    sv = sv_t.transpose(1, 0).reshape(bs, 1, _D)
    return (sv, nv)
